# Initial kernel scaffold; baseline (speedup 1.0000x reference)
import jax
import jax.numpy as jnp
from jax.experimental import pallas as pl


def kernel(preds, targets):
    raise NotImplementedError("write your pallas kernel here")


# speedup vs baseline: 1.5383x; 1.5383x over previous
import functools

import jax
import jax.numpy as jnp
from jax import lax
from jax.experimental import pallas as pl
from jax.experimental.pallas import tpu as pltpu
from jax.experimental.pallas import tpu_sc as plsc

N = 16384
T = 16
L = 16
NV = N // L
R = 2048
EPS = 1e-12


def _sc_zvalues(predsT, targetsT):
    mesh = plsc.VectorSubcoreMesh(core_axis_name="c", subcore_axis_name="s")

    @functools.partial(
        pl.kernel,
        out_type=jax.ShapeDtypeStruct((T, N), jnp.float32),
        mesh=mesh,
        compiler_params=pltpu.CompilerParams(needs_layout_passes=False),
        scratch_types=[
            pltpu.VMEM((N,), jnp.float32),
            pltpu.VMEM((N,), jnp.float32),
            pltpu.VMEM((N,), jnp.int32),
            pltpu.VMEM((N,), jnp.int32),
            pltpu.VMEM((N,), jnp.float32),
            pltpu.VMEM((R,), jnp.int32),
            pltpu.VMEM((N,), jnp.float32),
        ],
    )
    def k(predsT_hbm, targetsT_hbm, z_hbm, tgt_v, val_a, key_a, key_b, val_b,
          hist, out_v):
        c = lax.axis_index("c")
        s = lax.axis_index("s")

        @pl.when(s < 8)
        def _():
            task = s * 2 + c
            pltpu.sync_copy(targetsT_hbm.at[task], tgt_v)
            pltpu.sync_copy(predsT_hbm.at[task], val_a)

            occ0, _ = plsc.scan_count(jnp.zeros((L,), jnp.int32))
            base0 = jnp.min(occ0)

            def kb(i, mx):
                tv = tgt_v[pl.ds(i * L, L)]
                u = plsc.bitcast(tv, jnp.uint32)
                neg = (u >> 31) != 0
                key = jnp.where(neg, u, u ^ jnp.uint32(0x7FFFFFFF))
                key_a[pl.ds(i * L, L)] = plsc.bitcast(key, jnp.int32)
                return jnp.maximum(mx, val_a[pl.ds(i * L, L)])

            mx = lax.fori_loop(0, NV, kb,
                               jnp.full((L,), -jnp.inf, jnp.float32))
            smax = jnp.max(mx)

            def one_pass(shift, nbits, src_k, src_v, dst_k, dst_v):
                dmask = jnp.uint32((1 << nbits) - 1)

                def zh(j, _):
                    hist[pl.ds(j * L, L)] = jnp.zeros((L,), jnp.int32)
                    return 0

                lax.fori_loop(0, R // L, zh, 0)

                def hb(i, _):
                    kk = plsc.bitcast(src_k[pl.ds(i * L, L)], jnp.uint32)
                    d = ((kk >> jnp.uint32(shift)) & dmask).astype(jnp.int32)
                    occ, lastm = plsc.scan_count(d)
                    plsc.addupdate_scatter(hist, [d], occ - base0 + 1,
                                           mask=lastm)
                    return 0

                lax.fori_loop(0, NV, hb, 0)

                def sb(j, carry):
                    v = hist[pl.ds(j * L, L)]
                    cs = plsc.cumsum(v)
                    hist[pl.ds(j * L, L)] = cs - v + carry
                    return carry + jnp.max(cs)

                lax.fori_loop(0, R // L, sb, jnp.int32(0))

                def pb(i, _):
                    kk = src_k[pl.ds(i * L, L)]
                    vv = src_v[pl.ds(i * L, L)]
                    ku = plsc.bitcast(kk, jnp.uint32)
                    d = ((ku >> jnp.uint32(shift)) & dmask).astype(jnp.int32)
                    occ, lastm = plsc.scan_count(d)
                    base = plsc.load_gather(hist, [d])
                    pos = base + occ - base0
                    plsc.store_scatter(dst_k, [pos], kk)
                    plsc.store_scatter(dst_v, [pos], vv)
                    plsc.addupdate_scatter(hist, [d], occ - base0 + 1,
                                           mask=lastm)
                    return 0

                lax.fori_loop(0, NV, pb, 0)

            one_pass(0, 11, key_a, val_a, key_b, val_b)
            one_pass(11, 11, key_b, val_b, key_a, val_a)
            one_pass(22, 10, key_a, val_a, key_b, val_b)

            def suf(j, carry):
                i = NV - 1 - j
                v = val_b[pl.ds(i * L, L)]
                e = jnp.exp(v - smax)
                sfx = lax.rev(plsc.cumsum(lax.rev(e, (0,))), (0,)) + carry
                out_v[pl.ds(i * L, L)] = sfx
                return jnp.max(sfx)

            lax.fori_loop(0, NV, suf, jnp.float32(0.0))
            pltpu.sync_copy(out_v, z_hbm.at[task])

    return k(predsT, targetsT)


def _tc_finish(z, preds):
    def body(z_ref, p_ref, o_ref):
        lz = jnp.log(z_ref[...] + jnp.float32(EPS))
        o_ref[0, 0] = (jnp.sum(lz) - jnp.sum(p_ref[...])) / jnp.float32(N * T)

    out = pl.pallas_call(
        body,
        out_shape=jax.ShapeDtypeStruct((1, 1), jnp.float32),
        out_specs=pl.BlockSpec(memory_space=pltpu.SMEM),
    )(z, preds)
    return out[0, 0]


def kernel(preds, targets):
    predsT = preds.T
    targetsT = targets.T
    z = _sc_zvalues(predsT, targetsT)
    return _tc_finish(z, preds)


# speedup vs baseline: 1.6114x; 1.0475x over previous
import functools

import jax
import jax.numpy as jnp
from jax import lax
from jax.experimental import pallas as pl
from jax.experimental.pallas import tpu as pltpu
from jax.experimental.pallas import tpu_sc as plsc

N = 16384
T = 16
L = 16
NV = N // L
R = 2048
EPS = 1e-12


def _sc_zvalues(predsT, targetsT):
    mesh = plsc.VectorSubcoreMesh(core_axis_name="c", subcore_axis_name="s")

    @functools.partial(
        pl.kernel,
        out_type=jax.ShapeDtypeStruct((T, N), jnp.float32),
        mesh=mesh,
        compiler_params=pltpu.CompilerParams(needs_layout_passes=False),
        scratch_types=[
            pltpu.VMEM((N,), jnp.float32),
            pltpu.VMEM((N,), jnp.float32),
            pltpu.VMEM((N,), jnp.int32),
            pltpu.VMEM((N,), jnp.int32),
            pltpu.VMEM((N,), jnp.float32),
            pltpu.VMEM((R,), jnp.int32),
            pltpu.VMEM((N,), jnp.float32),
        ],
    )
    def k(predsT_hbm, targetsT_hbm, z_hbm, tgt_v, val_a, key_a, key_b, val_b,
          hist, out_v):
        c = lax.axis_index("c")
        s = lax.axis_index("s")

        @pl.when(s < 8)
        def _():
            task = s * 2 + c
            pltpu.sync_copy(targetsT_hbm.at[task], tgt_v)
            pltpu.sync_copy(predsT_hbm.at[task], val_a)

            occ0, _ = plsc.scan_count(jnp.zeros((L,), jnp.int32))
            base0 = jnp.min(occ0)

            def zh(j, _):
                hist[pl.ds(j * L, L)] = jnp.zeros((L,), jnp.int32)
                return 0

            lax.fori_loop(0, R // L, zh, 0, unroll=4)

            def kb(i, mx):
                tv = tgt_v[pl.ds(i * L, L)]
                u = plsc.bitcast(tv, jnp.uint32)
                neg = (u >> 31) != 0
                key = jnp.where(neg, u, u ^ jnp.uint32(0x7FFFFFFF))
                key_a[pl.ds(i * L, L)] = plsc.bitcast(key, jnp.int32)
                d = (key & jnp.uint32(0x7FF)).astype(jnp.int32)
                occ, lastm = plsc.scan_count(d)
                plsc.addupdate_scatter(hist, [d], occ - base0 + 1, mask=lastm)
                return jnp.maximum(mx, val_a[pl.ds(i * L, L)])

            mx = lax.fori_loop(0, NV, kb,
                               jnp.full((L,), -jnp.inf, jnp.float32),
                               unroll=4)
            smax = jnp.max(mx)

            def one_pass(shift, nbits, src_k, src_v, dst_k, dst_v,
                         skip_hist=False, exp_vals=False):
                dmask = jnp.uint32((1 << nbits) - 1)

                if not skip_hist:
                    def zh2(j, _):
                        hist[pl.ds(j * L, L)] = jnp.zeros((L,), jnp.int32)
                        return 0

                    lax.fori_loop(0, R // L, zh2, 0, unroll=4)

                    def hb(i, _):
                        kk = plsc.bitcast(src_k[pl.ds(i * L, L)], jnp.uint32)
                        d = ((kk >> jnp.uint32(shift)) & dmask).astype(
                            jnp.int32)
                        occ, lastm = plsc.scan_count(d)
                        plsc.addupdate_scatter(hist, [d], occ - base0 + 1,
                                               mask=lastm)
                        return 0

                    lax.fori_loop(0, NV, hb, 0, unroll=4)

                def sb(j, carry):
                    v = hist[pl.ds(j * L, L)]
                    cs = plsc.cumsum(v)
                    hist[pl.ds(j * L, L)] = cs - v + carry
                    return carry + jnp.max(cs)

                lax.fori_loop(0, R // L, sb, jnp.int32(0), unroll=4)

                def pb(i, _):
                    kk = src_k[pl.ds(i * L, L)]
                    vv = src_v[pl.ds(i * L, L)]
                    if exp_vals:
                        vv = jnp.exp(vv - smax)
                    ku = plsc.bitcast(kk, jnp.uint32)
                    d = ((ku >> jnp.uint32(shift)) & dmask).astype(jnp.int32)
                    occ, lastm = plsc.scan_count(d)
                    base = plsc.load_gather(hist, [d])
                    pos = base + occ - base0
                    if shift < 22:
                        plsc.store_scatter(dst_k, [pos], kk)
                    plsc.store_scatter(dst_v, [pos], vv)
                    plsc.addupdate_scatter(hist, [d], occ - base0 + 1,
                                           mask=lastm)
                    return 0

                lax.fori_loop(0, NV, pb, 0, unroll=4)

            one_pass(0, 11, key_a, val_a, key_b, val_b, skip_hist=True)
            one_pass(11, 11, key_b, val_b, key_a, val_a)
            one_pass(22, 10, key_a, val_a, key_b, val_b, exp_vals=True)

            def suf(j, carry):
                i = NV - 1 - j
                e = val_b[pl.ds(i * L, L)]
                sfx = lax.rev(plsc.cumsum(lax.rev(e, (0,))), (0,)) + carry
                out_v[pl.ds(i * L, L)] = sfx
                return jnp.max(sfx)

            lax.fori_loop(0, NV, suf, jnp.float32(0.0), unroll=4)
            pltpu.sync_copy(out_v, z_hbm.at[task])

    return k(predsT, targetsT)


def _tc_finish(z, preds):
    def body(z_ref, p_ref, o_ref):
        lz = jnp.log(z_ref[...] + jnp.float32(EPS))
        o_ref[0, 0] = (jnp.sum(lz) - jnp.sum(p_ref[...])) / jnp.float32(N * T)

    out = pl.pallas_call(
        body,
        out_shape=jax.ShapeDtypeStruct((1, 1), jnp.float32),
        out_specs=pl.BlockSpec(memory_space=pltpu.SMEM),
    )(z, preds)
    return out[0, 0]


def kernel(preds, targets):
    predsT = preds.T
    targetsT = targets.T
    z = _sc_zvalues(predsT, targetsT)
    return _tc_finish(z, preds)
